# baseline (speedup 1.0000x reference)
import functools

import jax
import jax.numpy as jnp
from jax import lax
from jax.experimental import pallas as pl
from jax.experimental.pallas import tpu as pltpu
from jax.experimental.pallas import tpu_sc as plsc

_VOCAB = 100000
_D = 128
_T = 4
_NL = 2
_TAG = 20
_B = 256
_S = 128

_NC = 2
_NS = 16
_NW = _NC * _NS
_ROWS = _B * _S
_RPW = _ROWS // _NW
_CH = 128
_NCH = _RPW // _CH


_NBUF = 4


def _sc_body(nch, row0, idx_hbm, table_hbm, out_hbm, idx_v, *scratch):
    rows = scratch[:_NBUF]
    gsems = scratch[_NBUF:2 * _NBUF]
    osems = scratch[2 * _NBUF:3 * _NBUF]
    wid = lax.axis_index("s") * _NC + lax.axis_index("c")
    pltpu.sync_copy(idx_hbm.at[pl.ds(row0 + wid * nch, nch)], idx_v)
    gh = [None] * _NBUF
    oh = [None] * _NBUF
    for c in range(min(2, nch)):
        gh[c] = pltpu.async_copy(table_hbm.at[idx_v.at[c]], rows[c], gsems[c])
    for c in range(nch):
        j = c + 2
        if j < nch:
            if j >= _NBUF:
                oh[j % _NBUF].wait()
            gh[j % _NBUF] = pltpu.async_copy(
                table_hbm.at[idx_v.at[j]], rows[j % _NBUF], gsems[j % _NBUF])
        gh[c % _NBUF].wait()
        oh[c % _NBUF] = pltpu.async_copy(
            rows[c % _NBUF], out_hbm.at[wid * nch + c], osems[c % _NBUF])
    for k in range(max(0, nch - _NBUF), nch):
        oh[k % _NBUF].wait()


@functools.lru_cache(maxsize=4)
def _make_emb_gather(nb, nch, row0):
    return pl.kernel(
        functools.partial(_sc_body, nch, row0),
        out_type=jax.ShapeDtypeStruct((nb, _S, _D), jnp.float32),
        mesh=plsc.VectorSubcoreMesh(core_axis_name="c", subcore_axis_name="s"),
        scratch_types=(
            [pltpu.VMEM((nch, _CH), jnp.int32)]
            + [pltpu.VMEM((_CH, _D), jnp.float32) for _ in range(_NBUF)]
            + [pltpu.SemaphoreType.DMA for _ in range(2 * _NBUF)]
        ),
    )


_BE = 32


def _sel_t(idx_row, sub):
    return (jnp.broadcast_to(idx_row, (_S, _S)) == sub).astype(jnp.float32)


def _gather_rows(idx_row, x, sub):
    pt = _sel_t(idx_row, sub)
    return lax.dot_general(pt, x, (((0,), (0,)), ((), ())))


def _tc_body(emb_ref, adj_ref, n2s_ref, s2n_ref, ws_ref, wg_ref, wt_ref, out_ref):
    sub = lax.broadcasted_iota(jnp.int32, (_S, _S), 0)
    ws = ws_ref[...]
    wt = wt_ref[...]
    emb_all = emb_ref[...].reshape(_BE * _S, _D)
    states_all = jax.nn.relu(jnp.dot(emb_all, ws))
    nodes = []
    for e in range(_BE):
        nodes.append(_gather_rows(n2s_ref[pl.ds(e, 1), :],
                                  states_all[e * _S:(e + 1) * _S], sub))
    node_all = jnp.concatenate(nodes, axis=0)
    for l in range(_NL):
        msgs = [jnp.dot(node_all, wg_ref[l * _T + t]) for t in range(_T)]
        aggs = []
        for e in range(_BE):
            r = jnp.concatenate([m[e * _S:(e + 1) * _S] for m in msgs], axis=0)
            aggs.append(jnp.dot(adj_ref[e], r))
        node_all = jax.nn.relu(node_all + jnp.concatenate(aggs, axis=0))
    seqs = []
    for e in range(_BE):
        seqs.append(_gather_rows(s2n_ref[pl.ds(e, 1), :],
                                 node_all[e * _S:(e + 1) * _S], sub))
    logits = jnp.dot(jnp.concatenate(seqs, axis=0), wt)
    big_lane = lax.broadcasted_iota(jnp.int32, (_BE * _S, _S), 1)
    masked = jnp.where(big_lane < _TAG, logits, -jnp.inf)
    m = jnp.max(masked, axis=1, keepdims=True)
    lse = m + jnp.log(jnp.sum(jnp.exp(masked - m), axis=1, keepdims=True))
    out_ref[...] = (masked - lse).reshape(_BE, _S, _S)


_H = 4
_BH = _B // _H


def _tc_half(h, emb_h, adj, n2s, s2n, ws, wg_flat, wt_pad):
    off = h * (_BH // _BE)
    return pl.pallas_call(
        _tc_body,
        grid=(_BH // _BE,),
        in_specs=[
            pl.BlockSpec((_BE, _S, _D), lambda i: (i, 0, 0)),
            pl.BlockSpec((_BE, _S, _T * _S), lambda i: (i + off, 0, 0)),
            pl.BlockSpec((_BE, _S), lambda i: (i + off, 0)),
            pl.BlockSpec((_BE, _S), lambda i: (i + off, 0)),
            pl.BlockSpec((_D, _D), lambda i: (0, 0)),
            pl.BlockSpec((_NL * _T, _D, _D), lambda i: (0, 0, 0)),
            pl.BlockSpec((_D, _S), lambda i: (0, 0)),
        ],
        out_specs=pl.BlockSpec((_BE, _S, _S), lambda i: (i, 0, 0)),
        out_shape=jax.ShapeDtypeStruct((_BH, _S, _S), jnp.float32),
    )(emb_h, adj, n2s, s2n, ws, wg_flat, wt_pad)


def kernel(wordSeqTensor, seq2NodeTensor, node2SeqTensor, adjMatrixTensor,
           emb_table, W_state, Wg, W_tag):
    idx = wordSeqTensor.astype(jnp.int32)
    n2s = node2SeqTensor.astype(jnp.int32).reshape(_B, _S)
    s2n = seq2NodeTensor.astype(jnp.int32).reshape(_B, _S)
    wg_flat = Wg.reshape(_NL * _T, _D, _D)
    wt_pad = jnp.pad(W_tag, ((0, 0), (0, _S - _TAG)))

    nch = _BH * _S // _NW // _CH
    outs = []
    for h in range(_H):
        emb_h = _make_emb_gather(_BH, nch, h * _BH)(idx, emb_table)
        outs.append(_tc_half(h, emb_h, adjMatrixTensor, n2s, s2n,
                             W_state, wg_flat, wt_pad))
    return jnp.concatenate(outs, axis=0)[:, :, :_TAG]


# speedup vs baseline: 1.1124x; 1.1124x over previous
import functools

import jax
import jax.numpy as jnp
from jax import lax
from jax.experimental import pallas as pl
from jax.experimental.pallas import tpu as pltpu
from jax.experimental.pallas import tpu_sc as plsc

_VOCAB = 100000
_D = 128
_T = 4
_NL = 2
_TAG = 20
_B = 256
_S = 128

_NC = 2
_NS = 16
_NW = _NC * _NS
_ROWS = _B * _S
_RPW = _ROWS // _NW
_CH = 128
_NCH = _RPW // _CH


_NBUF = 4


def _sc_body(nch, row0, idx_hbm, table_hbm, out_hbm, idx_v, *scratch):
    rows = scratch[:_NBUF]
    gsems = scratch[_NBUF:2 * _NBUF]
    osems = scratch[2 * _NBUF:3 * _NBUF]
    wid = lax.axis_index("s") * _NC + lax.axis_index("c")
    pltpu.sync_copy(idx_hbm.at[pl.ds(row0 + wid * nch, nch)], idx_v)
    gh = [None] * _NBUF
    oh = [None] * _NBUF
    for c in range(min(2, nch)):
        gh[c] = pltpu.async_copy(table_hbm.at[idx_v.at[c]], rows[c], gsems[c])
    for c in range(nch):
        j = c + 2
        if j < nch:
            if j >= _NBUF:
                oh[j % _NBUF].wait()
            gh[j % _NBUF] = pltpu.async_copy(
                table_hbm.at[idx_v.at[j]], rows[j % _NBUF], gsems[j % _NBUF])
        gh[c % _NBUF].wait()
        oh[c % _NBUF] = pltpu.async_copy(
            rows[c % _NBUF], out_hbm.at[wid * nch + c], osems[c % _NBUF])
    for k in range(max(0, nch - _NBUF), nch):
        oh[k % _NBUF].wait()


@functools.lru_cache(maxsize=4)
def _make_emb_gather(nb, nch, row0):
    return pl.kernel(
        functools.partial(_sc_body, nch, row0),
        out_type=jax.ShapeDtypeStruct((nb, _S, _D), jnp.float32),
        mesh=plsc.VectorSubcoreMesh(core_axis_name="c", subcore_axis_name="s"),
        scratch_types=(
            [pltpu.VMEM((nch, _CH), jnp.int32)]
            + [pltpu.VMEM((_CH, _D), jnp.float32) for _ in range(_NBUF)]
            + [pltpu.SemaphoreType.DMA for _ in range(2 * _NBUF)]
        ),
    )


_BE = 32


def _sel_t(idx_row, sub):
    return (jnp.broadcast_to(idx_row, (_S, _S)) == sub).astype(jnp.float32)


def _gather_rows(idx_row, x, sub):
    pt = _sel_t(idx_row, sub)
    return lax.dot_general(pt, x, (((0,), (0,)), ((), ())))


def _tc_body(emb_ref, adj_ref, n2s_ref, s2n_ref, ws_ref, wg_ref, wt_ref, out_ref):
    sub = lax.broadcasted_iota(jnp.int32, (_S, _S), 0)
    ws = ws_ref[...]
    wt = wt_ref[...]
    emb_all = emb_ref[...].reshape(_BE * _S, _D)
    states_all = jax.nn.relu(jnp.dot(emb_all, ws))
    nodes = []
    for e in range(_BE):
        nodes.append(_gather_rows(n2s_ref[pl.ds(e, 1), :],
                                  states_all[e * _S:(e + 1) * _S], sub))
    node_all = jnp.concatenate(nodes, axis=0)
    for l in range(_NL):
        msgs = [jnp.dot(node_all, wg_ref[l * _T + t]) for t in range(_T)]
        aggs = []
        for e in range(_BE):
            r = jnp.concatenate([m[e * _S:(e + 1) * _S] for m in msgs], axis=0)
            aggs.append(jnp.dot(adj_ref[e], r))
        node_all = jax.nn.relu(node_all + jnp.concatenate(aggs, axis=0))
    seqs = []
    for e in range(_BE):
        seqs.append(_gather_rows(s2n_ref[pl.ds(e, 1), :],
                                 node_all[e * _S:(e + 1) * _S], sub))
    logits = jnp.dot(jnp.concatenate(seqs, axis=0), wt)
    big_lane = lax.broadcasted_iota(jnp.int32, (_BE * _S, _S), 1)
    masked = jnp.where(big_lane < _TAG, logits, -jnp.inf)
    m = jnp.max(masked, axis=1, keepdims=True)
    lse = m + jnp.log(jnp.sum(jnp.exp(masked - m), axis=1, keepdims=True))
    out_ref[...] = (masked - lse).reshape(_BE, _S, _S)


_H = 2
_BH = _B // _H


def _tc_half(h, emb_h, adj, n2s, s2n, ws, wg_flat, wt_pad):
    off = h * (_BH // _BE)
    return pl.pallas_call(
        _tc_body,
        grid=(_BH // _BE,),
        in_specs=[
            pl.BlockSpec((_BE, _S, _D), lambda i: (i, 0, 0)),
            pl.BlockSpec((_BE, _S, _T * _S), lambda i: (i + off, 0, 0)),
            pl.BlockSpec((_BE, _S), lambda i: (i + off, 0)),
            pl.BlockSpec((_BE, _S), lambda i: (i + off, 0)),
            pl.BlockSpec((_D, _D), lambda i: (0, 0)),
            pl.BlockSpec((_NL * _T, _D, _D), lambda i: (0, 0, 0)),
            pl.BlockSpec((_D, _S), lambda i: (0, 0)),
        ],
        out_specs=pl.BlockSpec((_BE, _S, _S), lambda i: (i, 0, 0)),
        out_shape=jax.ShapeDtypeStruct((_BH, _S, _S), jnp.float32),
    )(emb_h, adj, n2s, s2n, ws, wg_flat, wt_pad)


def kernel(wordSeqTensor, seq2NodeTensor, node2SeqTensor, adjMatrixTensor,
           emb_table, W_state, Wg, W_tag):
    idx = wordSeqTensor.astype(jnp.int32)
    n2s = node2SeqTensor.astype(jnp.int32).reshape(_B, _S)
    s2n = seq2NodeTensor.astype(jnp.int32).reshape(_B, _S)
    wg_flat = Wg.reshape(_NL * _T, _D, _D)
    wt_pad = jnp.pad(W_tag, ((0, 0), (0, _S - _TAG)))

    nch = _BH * _S // _NW // _CH
    outs = []
    for h in range(_H):
        emb_h = _make_emb_gather(_BH, nch, h * _BH)(idx, emb_table)
        outs.append(_tc_half(h, emb_h, adjMatrixTensor, n2s, s2n,
                             W_state, wg_flat, wt_pad))
    return jnp.concatenate(outs, axis=0)[:, :, :_TAG]


# speedup vs baseline: 1.3333x; 1.1986x over previous
import functools

import jax
import jax.numpy as jnp
from jax import lax
from jax.experimental import pallas as pl
from jax.experimental.pallas import tpu as pltpu
from jax.experimental.pallas import tpu_sc as plsc

_VOCAB = 100000
_D = 128
_T = 4
_NL = 2
_TAG = 20
_B = 256
_S = 128

_NC = 2
_NS = 16
_NW = _NC * _NS
_ROWS = _B * _S
_RPW = _ROWS // _NW
_CH = 128
_NCH = _RPW // _CH


_NBUF = 4


def _sc_body(idx_hbm, table_hbm, out_hbm, idx_v, *scratch):
    rows = scratch[:_NBUF]
    gsems = scratch[_NBUF:2 * _NBUF]
    osems = scratch[2 * _NBUF:3 * _NBUF]
    wid = lax.axis_index("s") * _NC + lax.axis_index("c")
    pltpu.sync_copy(idx_hbm.at[pl.ds(wid * _NCH, _NCH)], idx_v)
    gh = [None] * _NBUF
    oh = [None] * _NBUF
    for c in range(min(2, _NCH)):
        gh[c] = pltpu.async_copy(table_hbm.at[idx_v.at[c]], rows[c], gsems[c])
    for c in range(_NCH):
        j = c + 2
        if j < _NCH:
            if j >= _NBUF:
                oh[j % _NBUF].wait()
            gh[j % _NBUF] = pltpu.async_copy(
                table_hbm.at[idx_v.at[j]], rows[j % _NBUF], gsems[j % _NBUF])
        gh[c % _NBUF].wait()
        oh[c % _NBUF] = pltpu.async_copy(
            rows[c % _NBUF], out_hbm.at[wid * _NCH + c], osems[c % _NBUF])
    for k in range(max(0, _NCH - _NBUF), _NCH):
        oh[k % _NBUF].wait()


@functools.lru_cache(maxsize=1)
def _make_emb_gather():
    return pl.kernel(
        _sc_body,
        out_type=jax.ShapeDtypeStruct((_B, _S, _D), jnp.float32),
        mesh=plsc.VectorSubcoreMesh(core_axis_name="c", subcore_axis_name="s"),
        scratch_types=(
            [pltpu.VMEM((_NCH, _CH), jnp.int32)]
            + [pltpu.VMEM((_CH, _D), jnp.float32) for _ in range(_NBUF)]
            + [pltpu.SemaphoreType.DMA for _ in range(2 * _NBUF)]
        ),
    )


_BE = 32
_OW = 32


def _sel_t(idx_row, sub):
    return (jnp.broadcast_to(idx_row, (_S, _S)) == sub).astype(jnp.float32)


def _gather_rows(idx_row, x, sub):
    pt = _sel_t(idx_row, sub)
    return lax.dot_general(pt, x, (((0,), (0,)), ((), ())))


def _tc_body(emb_ref, adj_ref, n2s_ref, s2n_ref, ws_ref, wg_ref, wt_ref, out_ref):
    sub = lax.broadcasted_iota(jnp.int32, (_S, _S), 0)
    ws = ws_ref[...]
    wt = wt_ref[...]
    emb_all = emb_ref[...].reshape(_BE * _S, _D)
    states_all = jax.nn.relu(jnp.dot(emb_all, ws))
    nodes = []
    for e in range(_BE):
        nodes.append(_gather_rows(n2s_ref[pl.ds(e, 1), :],
                                  states_all[e * _S:(e + 1) * _S], sub))
    node_all = jnp.concatenate(nodes, axis=0)
    for l in range(_NL):
        msgs = [jnp.dot(node_all, wg_ref[l * _T + t]) for t in range(_T)]
        aggs = []
        for e in range(_BE):
            r = jnp.concatenate([m[e * _S:(e + 1) * _S] for m in msgs], axis=0)
            aggs.append(jnp.dot(adj_ref[e], r))
        node_all = jax.nn.relu(node_all + jnp.concatenate(aggs, axis=0))
    seqs = []
    for e in range(_BE):
        seqs.append(_gather_rows(s2n_ref[pl.ds(e, 1), :],
                                 node_all[e * _S:(e + 1) * _S], sub))
    seq_all = jnp.concatenate(seqs, axis=0)
    logits_t = lax.dot_general(wt, seq_all, (((0,), (1,)), ((), ())))
    lt = logits_t[:_TAG]
    m_t = jnp.max(lt, axis=0, keepdims=True)
    s_t = jnp.sum(jnp.exp(lt - m_t), axis=0, keepdims=True)
    lse_t = m_t + jnp.log(s_t)
    out_t = logits_t[:_OW] - lse_t
    out_ref[...] = lax.transpose(out_t, (1, 0)).reshape(_BE, _S, _OW)


def kernel(wordSeqTensor, seq2NodeTensor, node2SeqTensor, adjMatrixTensor,
           emb_table, W_state, Wg, W_tag):
    idx = wordSeqTensor.astype(jnp.int32)
    emb_seq = _make_emb_gather()(idx, emb_table)

    n2s = node2SeqTensor.astype(jnp.int32).reshape(_B, _S)
    s2n = seq2NodeTensor.astype(jnp.int32).reshape(_B, _S)
    wg_flat = Wg.reshape(_NL * _T, _D, _D)
    wt_pad = jnp.pad(W_tag, ((0, 0), (0, _S - _TAG)))

    wide = pl.pallas_call(
        _tc_body,
        grid=(_B // _BE,),
        in_specs=[
            pl.BlockSpec((_BE, _S, _D), lambda i: (i, 0, 0)),
            pl.BlockSpec((_BE, _S, _T * _S), lambda i: (i, 0, 0)),
            pl.BlockSpec((_BE, _S), lambda i: (i, 0)),
            pl.BlockSpec((_BE, _S), lambda i: (i, 0)),
            pl.BlockSpec((_D, _D), lambda i: (0, 0)),
            pl.BlockSpec((_NL * _T, _D, _D), lambda i: (0, 0, 0)),
            pl.BlockSpec((_D, _S), lambda i: (0, 0)),
        ],
        out_specs=pl.BlockSpec((_BE, _S, _OW), lambda i: (i, 0, 0)),
        out_shape=jax.ShapeDtypeStruct((_B, _S, _OW), jnp.float32),
    )(emb_seq, adjMatrixTensor, n2s, s2n, W_state, wg_flat, wt_pad)
    return wide[:, :, :_TAG]


# speedup vs baseline: 1.3369x; 1.0027x over previous
import functools

import jax
import jax.numpy as jnp
from jax import lax
from jax.experimental import pallas as pl
from jax.experimental.pallas import tpu as pltpu
from jax.experimental.pallas import tpu_sc as plsc

_VOCAB = 100000
_D = 128
_T = 4
_NL = 2
_TAG = 20
_B = 256
_S = 128

_NC = 2
_NS = 16
_NW = _NC * _NS
_ROWS = _B * _S
_RPW = _ROWS // _NW
_CH = 128
_NCH = _RPW // _CH


_NBUF = 6


def _sc_body(idx_hbm, table_hbm, out_hbm, idx_v, *scratch):
    rows = scratch[:_NBUF]
    gsems = scratch[_NBUF:2 * _NBUF]
    osems = scratch[2 * _NBUF:3 * _NBUF]
    wid = lax.axis_index("s") * _NC + lax.axis_index("c")
    pltpu.sync_copy(idx_hbm.at[pl.ds(wid * _NCH, _NCH)], idx_v)
    gh = [None] * _NBUF
    oh = [None] * _NBUF
    for c in range(min(3, _NCH)):
        gh[c] = pltpu.async_copy(table_hbm.at[idx_v.at[c]], rows[c], gsems[c])
    for c in range(_NCH):
        j = c + 3
        if j < _NCH:
            if j >= _NBUF:
                oh[j % _NBUF].wait()
            gh[j % _NBUF] = pltpu.async_copy(
                table_hbm.at[idx_v.at[j]], rows[j % _NBUF], gsems[j % _NBUF])
        gh[c % _NBUF].wait()
        oh[c % _NBUF] = pltpu.async_copy(
            rows[c % _NBUF], out_hbm.at[wid * _NCH + c], osems[c % _NBUF])
    for k in range(max(0, _NCH - _NBUF), _NCH):
        oh[k % _NBUF].wait()


@functools.lru_cache(maxsize=1)
def _make_emb_gather():
    return pl.kernel(
        _sc_body,
        out_type=jax.ShapeDtypeStruct((_B, _S, _D), jnp.float32),
        mesh=plsc.VectorSubcoreMesh(core_axis_name="c", subcore_axis_name="s"),
        scratch_types=(
            [pltpu.VMEM((_NCH, _CH), jnp.int32)]
            + [pltpu.VMEM((_CH, _D), jnp.float32) for _ in range(_NBUF)]
            + [pltpu.SemaphoreType.DMA for _ in range(2 * _NBUF)]
        ),
    )


_BE = 32
_OW = 32


def _sel_t(idx_row, sub):
    return (jnp.broadcast_to(idx_row, (_S, _S)) == sub).astype(jnp.float32)


def _gather_rows(idx_row, x, sub):
    pt = _sel_t(idx_row, sub)
    return lax.dot_general(pt, x, (((0,), (0,)), ((), ())))


def _tc_body(emb_ref, adj_ref, n2s_ref, s2n_ref, ws_ref, wg_ref, wt_ref, out_ref):
    sub = lax.broadcasted_iota(jnp.int32, (_S, _S), 0)
    ws = ws_ref[...]
    wt = wt_ref[...]
    emb_all = emb_ref[...].reshape(_BE * _S, _D)
    states_all = jax.nn.relu(jnp.dot(emb_all, ws))
    nodes = []
    for e in range(_BE):
        nodes.append(_gather_rows(n2s_ref[pl.ds(e, 1), :],
                                  states_all[e * _S:(e + 1) * _S], sub))
    node_all = jnp.concatenate(nodes, axis=0)
    for l in range(_NL):
        msgs = [jnp.dot(node_all, wg_ref[l * _T + t]) for t in range(_T)]
        aggs = []
        for e in range(_BE):
            r = jnp.concatenate([m[e * _S:(e + 1) * _S] for m in msgs], axis=0)
            aggs.append(jnp.dot(adj_ref[e], r))
        node_all = jax.nn.relu(node_all + jnp.concatenate(aggs, axis=0))
    seqs = []
    for e in range(_BE):
        seqs.append(_gather_rows(s2n_ref[pl.ds(e, 1), :],
                                 node_all[e * _S:(e + 1) * _S], sub))
    seq_all = jnp.concatenate(seqs, axis=0)
    logits_t = lax.dot_general(wt, seq_all, (((0,), (1,)), ((), ())))
    lt = logits_t[:_TAG]
    m_t = jnp.max(lt, axis=0, keepdims=True)
    s_t = jnp.sum(jnp.exp(lt - m_t), axis=0, keepdims=True)
    lse_t = m_t + jnp.log(s_t)
    out_t = logits_t[:_OW] - lse_t
    out_ref[...] = lax.transpose(out_t, (1, 0)).reshape(_BE, _S, _OW)


def kernel(wordSeqTensor, seq2NodeTensor, node2SeqTensor, adjMatrixTensor,
           emb_table, W_state, Wg, W_tag):
    idx = wordSeqTensor.astype(jnp.int32)
    emb_seq = _make_emb_gather()(idx, emb_table)

    n2s = node2SeqTensor.astype(jnp.int32).reshape(_B, _S)
    s2n = seq2NodeTensor.astype(jnp.int32).reshape(_B, _S)
    wg_flat = Wg.reshape(_NL * _T, _D, _D)
    wt_pad = jnp.pad(W_tag, ((0, 0), (0, _S - _TAG)))

    wide = pl.pallas_call(
        _tc_body,
        grid=(_B // _BE,),
        in_specs=[
            pl.BlockSpec((_BE, _S, _D), lambda i: (i, 0, 0)),
            pl.BlockSpec((_BE, _S, _T * _S), lambda i: (i, 0, 0)),
            pl.BlockSpec((_BE, _S), lambda i: (i, 0)),
            pl.BlockSpec((_BE, _S), lambda i: (i, 0)),
            pl.BlockSpec((_D, _D), lambda i: (0, 0)),
            pl.BlockSpec((_NL * _T, _D, _D), lambda i: (0, 0, 0)),
            pl.BlockSpec((_D, _S), lambda i: (0, 0)),
        ],
        out_specs=pl.BlockSpec((_BE, _S, _OW), lambda i: (i, 0, 0)),
        out_shape=jax.ShapeDtypeStruct((_B, _S, _OW), jnp.float32),
    )(emb_seq, adjMatrixTensor, n2s, s2n, W_state, wg_flat, wt_pad)
    return wide[:, :, :_TAG]
